# Initial kernel scaffold; baseline (speedup 1.0000x reference)
import jax
import jax.numpy as jnp
from jax.experimental import pallas as pl


def kernel(particles, weights, edges):
    raise NotImplementedError("write your pallas kernel here")


# speedup vs baseline: 3.7138x; 3.7138x over previous
import functools

import jax
import jax.numpy as jnp
from jax import lax
from jax.experimental import pallas as pl
from jax.experimental.pallas import tpu as pltpu
from jax.experimental.pallas import tpu_sc as plsc

N_NODES = 50000
N_EDGES = 800000
P = 8
D = 4 * P
NC = 2
NS = 16
NW = NC * NS
L = 16

EPW = N_EDGES // NW
C = 1000
CPAD = 1024
NCHUNK = EPW // C
GB = 128
NGATH = CPAD // GB
NG = CPAD // L


def _splat(v):
    return jnp.full((L,), v, dtype=jnp.int32)


def _fori(n, body):
    lax.fori_loop(jnp.int32(0), jnp.int32(n), body, jnp.int32(0))


def _sc_body(ptab, ei, ej, w16, ratios, rmw,
             ei_v, ej_v, xi_v, xj_v, out_v, w_v, w16_v, sem_i, sem_j):
    wid = lax.axis_index("s") * NC + lax.axis_index("c")
    zero16 = jnp.zeros((L,), dtype=jnp.int32)
    iota16 = lax.iota(jnp.int32, L)

    def zinit(i, carry):
        o = i * jnp.int32(L)
        ei_v[pl.ds(o, L)] = zero16
        ej_v[pl.ds(o, L)] = zero16
        return carry
    _fori(CPAD // L, zinit)

    pltpu.sync_copy(w16, w16_v)
    wpat = w16_v[...]

    def wfill(i, carry):
        w_v[pl.ds(i * jnp.int32(L), L)] = wpat
        return carry
    _fori((C * P) // L, wfill)

    def chunk(k, carry):
        base = wid * jnp.int32(EPW) + k * jnp.int32(C)
        pltpu.sync_copy(ei.at[pl.ds(base, C)], ei_v.at[pl.ds(0, C)])
        pltpu.sync_copy(ej.at[pl.ds(base, C)], ej_v.at[pl.ds(0, C)])
        copies = []
        for j in range(NGATH):
            s = j * GB
            copies.append(pltpu.async_copy(
                ptab.at[ei_v.at[pl.ds(s, GB)]], xi_v.at[pl.ds(s, GB)], sem_i))
            copies.append(pltpu.async_copy(
                ptab.at[ej_v.at[pl.ds(s, GB)]], xj_v.at[pl.ds(s, GB)], sem_j))
        for cp in copies:
            cp.wait()

        def grp(g, inner):
            e16 = g * jnp.int32(L) + iota16
            for p in range(P):
                b = 4 * p
                w1 = plsc.load_gather(xi_v, [e16, _splat(b)])
                x1 = plsc.load_gather(xi_v, [e16, _splat(b + 1)])
                y1 = plsc.load_gather(xi_v, [e16, _splat(b + 2)])
                z1 = plsc.load_gather(xi_v, [e16, _splat(b + 3)])
                w2 = plsc.load_gather(xj_v, [e16, _splat(b)])
                x2 = plsc.load_gather(xj_v, [e16, _splat(b + 1)])
                y2 = plsc.load_gather(xj_v, [e16, _splat(b + 2)])
                z2 = plsc.load_gather(xj_v, [e16, _splat(b + 3)])
                rw = (w1 * w2 + x1 * x2) + (y1 * y2 + z1 * z2)
                rx = (x1 * w2 - w1 * x2) + (z1 * y2 - y1 * z2)
                ry = (y1 * w2 - w1 * y2) + (x1 * z2 - z1 * x2)
                rz = (z1 * w2 - w1 * z2) + (y1 * x2 - x1 * y2)
                plsc.store_scatter(out_v, [e16, _splat(b)], rw)
                plsc.store_scatter(out_v, [e16, _splat(b + 1)], rx)
                plsc.store_scatter(out_v, [e16, _splat(b + 2)], ry)
                plsc.store_scatter(out_v, [e16, _splat(b + 3)], rz)
            return inner
        _fori(NG, grp)

        pltpu.sync_copy(out_v.at[pl.ds(0, C)], ratios.at[pl.ds(base, C)])
        pltpu.sync_copy(w_v.at[pl.ds(0, C * P)],
                        rmw.at[pl.ds(base * jnp.int32(P), C * P)])
        return carry
    _fori(NCHUNK, chunk)


@functools.partial(
    pl.kernel,
    out_type=(jax.ShapeDtypeStruct((N_EDGES, D), jnp.float32),
              jax.ShapeDtypeStruct((N_EDGES * P,), jnp.float32)),
    mesh=plsc.VectorSubcoreMesh(core_axis_name="c", subcore_axis_name="s",
                                num_cores=NC, num_subcores=NS),
    compiler_params=pltpu.CompilerParams(needs_layout_passes=False,
                                         use_tc_tiling_on_sc=False),
    scratch_types=[
        pltpu.VMEM((CPAD,), jnp.int32),
        pltpu.VMEM((CPAD,), jnp.int32),
        pltpu.VMEM((CPAD, D), jnp.float32),
        pltpu.VMEM((CPAD, D), jnp.float32),
        pltpu.VMEM((CPAD, D), jnp.float32),
        pltpu.VMEM((C * P,), jnp.float32),
        pltpu.VMEM((L,), jnp.float32),
        pltpu.SemaphoreType.DMA,
        pltpu.SemaphoreType.DMA,
    ],
)
def _quat_edges_sc(ptab, ei, ej, w16, ratios, rmw, *scratch):
    _sc_body(ptab, ei, ej, w16, ratios, rmw, *scratch)


def kernel(particles, weights, edges):
    ptab = particles.astype(jnp.float32).reshape(N_NODES, D)
    ei = edges[:, 0].astype(jnp.int32)
    ej = edges[:, 1].astype(jnp.int32)
    w16 = jnp.tile(weights.astype(jnp.float32).reshape(-1), 2)
    ratios, rmw = _quat_edges_sc(ptab, ei, ej, w16)
    return ratios.reshape(N_EDGES, P, 4), rmw.reshape(N_EDGES, P)
